# Initial kernel scaffold; baseline (speedup 1.0000x reference)
import jax
import jax.numpy as jnp
from jax.experimental import pallas as pl


def kernel(z, embeddings):
    raise NotImplementedError("write your pallas kernel here")


# speedup vs baseline: 1.6747x; 1.6747x over previous
import functools

import jax
import jax.numpy as jnp
from jax.experimental import pallas as pl

NUM_CODES = 1024
DIM = 64
S_BLK = 512


def _vq_body(z_ref, zsq_ref, emb_ref, e2_ref, q_ref, idx_ref, loss_ref):
    zb = z_ref[0]
    prod = jax.lax.dot_general(
        emb_ref[...], zb, (((1,), (0,)), ((), ())),
        preferred_element_type=jnp.float32)
    d = (zsq_ref[0] + e2_ref[...]) - 2.0 * prod
    minval = jnp.min(d, axis=0)
    rows = jax.lax.broadcasted_iota(jnp.int32, (NUM_CODES, S_BLK), 0)
    idx = jnp.min(jnp.where(d == minval[None, :], rows, NUM_CODES), axis=0)
    idx_ref[0, 0, :] = idx

    onehot = (rows == idx[None, :]).astype(jnp.float32)
    q_ref[0] = jax.lax.dot_general(
        emb_ref[...], onehot, (((0,), (0,)), ((), ())),
        preferred_element_type=jnp.float32)

    part = jnp.sum(minval).reshape(1, 1)
    first = (pl.program_id(0) == 0) & (pl.program_id(1) == 0)

    @pl.when(first)
    def _():
        loss_ref[...] = part

    @pl.when(jnp.logical_not(first))
    def _():
        loss_ref[...] += part


@functools.partial(jax.jit, static_argnames=())
def kernel(z, embeddings):
    B, C, T, H, W = z.shape
    S = T * H * W
    nblk = S // S_BLK
    z3 = z.reshape(B, C, S)
    zf = jnp.transpose(z, (0, 2, 3, 4, 1)).reshape(-1, C)
    zsq = jnp.sum(zf ** 2, axis=1).reshape(B, 1, S)
    e2 = jnp.sum(embeddings ** 2, axis=1)[:, None]

    q3, idx3, loss_sum = pl.pallas_call(
        _vq_body,
        grid=(B, nblk),
        in_specs=[
            pl.BlockSpec((1, C, S_BLK), lambda b, j: (b, 0, j)),
            pl.BlockSpec((1, 1, S_BLK), lambda b, j: (b, 0, j)),
            pl.BlockSpec((NUM_CODES, DIM), lambda b, j: (0, 0)),
            pl.BlockSpec((NUM_CODES, 1), lambda b, j: (0, 0)),
        ],
        out_specs=[
            pl.BlockSpec((1, C, S_BLK), lambda b, j: (b, 0, j)),
            pl.BlockSpec((1, 1, S_BLK), lambda b, j: (b * nblk + j, 0, 0)),
            pl.BlockSpec((1, 1), lambda b, j: (0, 0)),
        ],
        out_shape=[
            jax.ShapeDtypeStruct((B, C, S), jnp.float32),
            jax.ShapeDtypeStruct((B * nblk, 1, S_BLK), jnp.int32),
            jax.ShapeDtypeStruct((1, 1), jnp.float32),
        ],
    )(z3, zsq, embeddings, e2)

    quantized_st = q3.reshape(B, C, T, H, W)
    encoding_indices = idx3.reshape(B, T, H, W)
    vq_loss = (1.0 + 0.25) * loss_sum[0, 0] / z.size
    return (quantized_st, vq_loss, encoding_indices)


# speedup vs baseline: 1.9538x; 1.1667x over previous
import functools

import jax
import jax.numpy as jnp
from jax.experimental import pallas as pl

NUM_CODES = 1024
DIM = 64
S_BLK = 1024


def _vq_body(z_ref, zsq_ref, emb_ref, neg2emb_ref, e2_ref, q_ref, idx_ref,
             loss_ref):
    zb = z_ref[0]
    prod = jax.lax.dot_general(
        neg2emb_ref[...], zb, (((1,), (0,)), ((), ())),
        preferred_element_type=jnp.float32)
    d = (zsq_ref[0] + e2_ref[...]) + prod
    minval = jnp.min(d, axis=0)
    rows = jax.lax.broadcasted_iota(jnp.int32, (NUM_CODES, S_BLK), 0)
    idx = jnp.min(jnp.where(d == minval[None, :], rows, NUM_CODES), axis=0)
    idx_ref[0, 0, :] = idx

    onehot = (rows == idx[None, :]).astype(jnp.float32)
    q_ref[0] = jax.lax.dot_general(
        emb_ref[...], onehot, (((0,), (0,)), ((), ())),
        preferred_element_type=jnp.float32)

    part = jnp.sum(minval).reshape(1, 1)
    first = (pl.program_id(0) == 0) & (pl.program_id(1) == 0)

    @pl.when(first)
    def _():
        loss_ref[...] = part

    @pl.when(jnp.logical_not(first))
    def _():
        loss_ref[...] += part


@functools.partial(jax.jit, static_argnames=())
def kernel(z, embeddings):
    B, C, T, H, W = z.shape
    S = T * H * W
    nblk = S // S_BLK
    z3 = z.reshape(B, C, S)
    zf = jnp.transpose(z, (0, 2, 3, 4, 1)).reshape(-1, C)
    zsq = jnp.sum(zf ** 2, axis=1).reshape(B, 1, S)
    e2 = jnp.sum(embeddings ** 2, axis=1)[:, None]
    neg2emb = -2.0 * embeddings

    q3, idx3, loss_sum = pl.pallas_call(
        _vq_body,
        grid=(B, nblk),
        in_specs=[
            pl.BlockSpec((1, C, S_BLK), lambda b, j: (b, 0, j)),
            pl.BlockSpec((1, 1, S_BLK), lambda b, j: (b, 0, j)),
            pl.BlockSpec((NUM_CODES, DIM), lambda b, j: (0, 0)),
            pl.BlockSpec((NUM_CODES, DIM), lambda b, j: (0, 0)),
            pl.BlockSpec((NUM_CODES, 1), lambda b, j: (0, 0)),
        ],
        out_specs=[
            pl.BlockSpec((1, C, S_BLK), lambda b, j: (b, 0, j)),
            pl.BlockSpec((1, 1, S_BLK), lambda b, j: (b * nblk + j, 0, 0)),
            pl.BlockSpec((1, 1), lambda b, j: (0, 0)),
        ],
        out_shape=[
            jax.ShapeDtypeStruct((B, C, S), jnp.float32),
            jax.ShapeDtypeStruct((B * nblk, 1, S_BLK), jnp.int32),
            jax.ShapeDtypeStruct((1, 1), jnp.float32),
        ],
    )(z3, zsq, embeddings, neg2emb, e2)

    quantized_st = q3.reshape(B, C, T, H, W)
    encoding_indices = idx3.reshape(B, T, H, W)
    vq_loss = (1.0 + 0.25) * loss_sum[0, 0] / z.size
    return (quantized_st, vq_loss, encoding_indices)


# speedup vs baseline: 2.0790x; 1.0641x over previous
import functools

import jax
import jax.numpy as jnp
from jax.experimental import pallas as pl

NUM_CODES = 1024
DIM = 64
S_BLK = 2048


def _vq_body(z_ref, zsq_ref, emb_ref, neg2emb_ref, e2_ref, q_ref, idx_ref,
             loss_ref):
    zb = z_ref[0]
    zsq = zsq_ref[0]
    prod = jax.lax.dot_general(
        neg2emb_ref[...], zb, (((1,), (0,)), ((), ())),
        preferred_element_type=jnp.float32)
    d = (zsq + e2_ref[...]) + prod
    minval = jnp.min(d, axis=0)
    rows = jax.lax.broadcasted_iota(jnp.int32, (NUM_CODES, S_BLK), 0)
    idx = jnp.min(jnp.where(d == minval[None, :], rows, NUM_CODES), axis=0)
    idx_ref[0, 0, :] = idx

    onehot = (rows == idx[None, :]).astype(jnp.float32)
    q_ref[0] = jax.lax.dot_general(
        emb_ref[...], onehot, (((0,), (0,)), ((), ())),
        preferred_element_type=jnp.float32)

    part = jnp.sum(minval).reshape(1, 1)
    first = (pl.program_id(0) == 0) & (pl.program_id(1) == 0)

    @pl.when(first)
    def _():
        loss_ref[...] = part

    @pl.when(jnp.logical_not(first))
    def _():
        loss_ref[...] += part


@functools.partial(jax.jit, static_argnames=())
def kernel(z, embeddings):
    B, C, T, H, W = z.shape
    S = T * H * W
    nblk = S // S_BLK
    z3 = z.reshape(B, C, S)
    zf = jnp.transpose(z, (0, 2, 3, 4, 1)).reshape(-1, C)
    zsq = jnp.sum(zf ** 2, axis=1).reshape(B, 1, S)
    e2 = jnp.sum(embeddings ** 2, axis=1)[:, None]
    neg2emb = -2.0 * embeddings

    q3, idx3, loss_sum = pl.pallas_call(
        _vq_body,
        grid=(B, nblk),
        in_specs=[
            pl.BlockSpec((1, C, S_BLK), lambda b, j: (b, 0, j)),
            pl.BlockSpec((1, 1, S_BLK), lambda b, j: (b, 0, j)),
            pl.BlockSpec((NUM_CODES, DIM), lambda b, j: (0, 0)),
            pl.BlockSpec((NUM_CODES, DIM), lambda b, j: (0, 0)),
            pl.BlockSpec((NUM_CODES, 1), lambda b, j: (0, 0)),
        ],
        out_specs=[
            pl.BlockSpec((1, C, S_BLK), lambda b, j: (b, 0, j)),
            pl.BlockSpec((1, 1, S_BLK), lambda b, j: (b * nblk + j, 0, 0)),
            pl.BlockSpec((1, 1), lambda b, j: (0, 0)),
        ],
        out_shape=[
            jax.ShapeDtypeStruct((B, C, S), jnp.float32),
            jax.ShapeDtypeStruct((B * nblk, 1, S_BLK), jnp.int32),
            jax.ShapeDtypeStruct((1, 1), jnp.float32),
        ],
    )(z3, zsq, embeddings, neg2emb, e2)

    quantized_st = q3.reshape(B, C, T, H, W)
    encoding_indices = idx3.reshape(B, T, H, W)
    vq_loss = (1.0 + 0.25) * loss_sum[0, 0] / z.size
    return (quantized_st, vq_loss, encoding_indices)


# speedup vs baseline: 2.5452x; 1.2243x over previous
import functools

import jax
import jax.numpy as jnp
from jax.experimental import pallas as pl

NUM_CODES = 1024
DIM = 64
S_BLK = 2048


def _vq_body(z_ref, zsq_ref, emb_ref, neg2emb_ref, e2_ref, q_ref, idx_ref,
             loss_ref):
    zb = z_ref[0]
    zsq = zsq_ref[0]
    prod = jax.lax.dot_general(
        neg2emb_ref[...], zb, (((1,), (0,)), ((), ())),
        preferred_element_type=jnp.float32)
    e2b = e2_ref[...]
    cur_min = (zsq + e2b[0:8]) + prod[0:8]
    cur_row = jnp.zeros((8, S_BLK), jnp.int32)
    for r in range(1, NUM_CODES // 8):
        dr = (zsq + e2b[8 * r:8 * (r + 1)]) + prod[8 * r:8 * (r + 1)]
        lt = dr < cur_min
        cur_min = jnp.where(lt, dr, cur_min)
        cur_row = jnp.where(lt, r, cur_row)
    sub = jax.lax.broadcasted_iota(jnp.int32, (8, S_BLK), 0)
    idx8 = cur_row * 8 + sub
    minval = jnp.min(cur_min, axis=0)
    idx = jnp.min(
        jnp.where(cur_min == minval[None, :], idx8, NUM_CODES), axis=0)
    idx_ref[0, 0, :] = idx

    rows = jax.lax.broadcasted_iota(jnp.int32, (NUM_CODES, S_BLK), 0)
    onehot = (rows == idx[None, :]).astype(jnp.float32)
    q_ref[0] = jax.lax.dot_general(
        emb_ref[...], onehot, (((0,), (0,)), ((), ())),
        preferred_element_type=jnp.float32)

    part = jnp.sum(minval).reshape(1, 1)
    first = (pl.program_id(0) == 0) & (pl.program_id(1) == 0)

    @pl.when(first)
    def _():
        loss_ref[...] = part

    @pl.when(jnp.logical_not(first))
    def _():
        loss_ref[...] += part


@functools.partial(jax.jit, static_argnames=())
def kernel(z, embeddings):
    B, C, T, H, W = z.shape
    S = T * H * W
    nblk = S // S_BLK
    z3 = z.reshape(B, C, S)
    zf = jnp.transpose(z, (0, 2, 3, 4, 1)).reshape(-1, C)
    zsq = jnp.sum(zf ** 2, axis=1).reshape(B, 1, S)
    e2 = jnp.sum(embeddings ** 2, axis=1)[:, None]
    neg2emb = -2.0 * embeddings

    q3, idx3, loss_sum = pl.pallas_call(
        _vq_body,
        grid=(B, nblk),
        in_specs=[
            pl.BlockSpec((1, C, S_BLK), lambda b, j: (b, 0, j)),
            pl.BlockSpec((1, 1, S_BLK), lambda b, j: (b, 0, j)),
            pl.BlockSpec((NUM_CODES, DIM), lambda b, j: (0, 0)),
            pl.BlockSpec((NUM_CODES, DIM), lambda b, j: (0, 0)),
            pl.BlockSpec((NUM_CODES, 1), lambda b, j: (0, 0)),
        ],
        out_specs=[
            pl.BlockSpec((1, C, S_BLK), lambda b, j: (b, 0, j)),
            pl.BlockSpec((1, 1, S_BLK), lambda b, j: (b * nblk + j, 0, 0)),
            pl.BlockSpec((1, 1), lambda b, j: (0, 0)),
        ],
        out_shape=[
            jax.ShapeDtypeStruct((B, C, S), jnp.float32),
            jax.ShapeDtypeStruct((B * nblk, 1, S_BLK), jnp.int32),
            jax.ShapeDtypeStruct((1, 1), jnp.float32),
        ],
    )(z3, zsq, embeddings, neg2emb, e2)

    quantized_st = q3.reshape(B, C, T, H, W)
    encoding_indices = idx3.reshape(B, T, H, W)
    vq_loss = (1.0 + 0.25) * loss_sum[0, 0] / z.size
    return (quantized_st, vq_loss, encoding_indices)


# speedup vs baseline: 2.5476x; 1.0009x over previous
import functools

import jax
import jax.numpy as jnp
from jax.experimental import pallas as pl

NUM_CODES = 1024
DIM = 64
S_BLK = 2048
CHUNK = 512


def _vq_body(z_ref, zsq_ref, emb_ref, neg2emb_ref, e2_ref,
             q_ref, idx_ref, loss_ref):
    zb = z_ref[0]
    zsq = zsq_ref[0]
    prod = jax.lax.dot_general(
        neg2emb_ref[...], zb, (((1,), (0,)), ((), ())),
        preferred_element_type=jnp.float32)
    e2b = e2_ref[...]
    sub = jax.lax.broadcasted_iota(jnp.int32, (8, CHUNK), 0)
    idx_parts = []
    minval_parts = []
    for c0 in range(0, S_BLK, CHUNK):
        zs = zsq[:, c0:c0 + CHUNK]
        cur_min = (zs + e2b[0:8]) + prod[0:8, c0:c0 + CHUNK]
        cur_row = jnp.zeros((8, CHUNK), jnp.int32)
        for r in range(1, NUM_CODES // 8):
            dr = (zs + e2b[8 * r:8 * (r + 1)]) + prod[8 * r:8 * (r + 1),
                                                      c0:c0 + CHUNK]
            lt = dr < cur_min
            cur_min = jnp.where(lt, dr, cur_min)
            cur_row = jnp.where(lt, r, cur_row)
        idx8 = cur_row * 8 + sub
        mv = jnp.min(cur_min, axis=0)
        idx_parts.append(jnp.min(
            jnp.where(cur_min == mv[None, :], idx8, NUM_CODES), axis=0))
        minval_parts.append(mv)
    idx = jnp.concatenate(idx_parts)
    minval = jnp.concatenate(minval_parts)
    idx_ref[0, 0, :] = idx

    rows = jax.lax.broadcasted_iota(jnp.int32, (NUM_CODES, S_BLK), 0)
    onehot = (rows == idx[None, :]).astype(jnp.float32)
    q_ref[0] = jax.lax.dot_general(
        emb_ref[...], onehot, (((0,), (0,)), ((), ())),
        preferred_element_type=jnp.float32)

    part = jnp.sum(minval).reshape(1, 1)
    first = (pl.program_id(0) == 0) & (pl.program_id(1) == 0)

    @pl.when(first)
    def _():
        loss_ref[...] = part

    @pl.when(jnp.logical_not(first))
    def _():
        loss_ref[...] += part


@functools.partial(jax.jit, static_argnames=())
def kernel(z, embeddings):
    B, C, T, H, W = z.shape
    S = T * H * W
    nblk = S // S_BLK
    z3 = z.reshape(B, C, S)
    zsq = jnp.sum(z3 * z3, axis=1).reshape(B, 1, S)
    e2 = jnp.sum(embeddings ** 2, axis=1)[:, None]
    neg2emb = -2.0 * embeddings

    q3, idx3, loss_sum = pl.pallas_call(
        _vq_body,
        grid=(B, nblk),
        in_specs=[
            pl.BlockSpec((1, C, S_BLK), lambda b, j: (b, 0, j)),
            pl.BlockSpec((1, 1, S_BLK), lambda b, j: (b, 0, j)),
            pl.BlockSpec((NUM_CODES, DIM), lambda b, j: (0, 0)),
            pl.BlockSpec((NUM_CODES, DIM), lambda b, j: (0, 0)),
            pl.BlockSpec((NUM_CODES, 1), lambda b, j: (0, 0)),
        ],
        out_specs=[
            pl.BlockSpec((1, C, S_BLK), lambda b, j: (b, 0, j)),
            pl.BlockSpec((1, 1, S_BLK), lambda b, j: (b * nblk + j, 0, 0)),
            pl.BlockSpec((1, 1), lambda b, j: (0, 0)),
        ],
        out_shape=[
            jax.ShapeDtypeStruct((B, C, S), jnp.float32),
            jax.ShapeDtypeStruct((B * nblk, 1, S_BLK), jnp.int32),
            jax.ShapeDtypeStruct((1, 1), jnp.float32),
        ],
    )(z3, zsq, embeddings, neg2emb, e2)

    quantized_st = q3.reshape(B, C, T, H, W)
    encoding_indices = idx3.reshape(B, T, H, W)
    vq_loss = (1.0 + 0.25) * loss_sum[0, 0] / z.size
    return (quantized_st, vq_loss, encoding_indices)


# speedup vs baseline: 2.8357x; 1.1131x over previous
import functools

import jax
import jax.numpy as jnp
from jax.experimental import pallas as pl

NUM_CODES = 1024
DIM = 64
S_BLK = 2048
CHUNK = 512


def _vq_body(z_ref, emb_ref, neg2emb_ref, e2_ref,
             q_ref, idx_ref, loss_ref):
    zb = z_ref[0]
    v = zb * zb
    for half in (32, 16, 8, 4, 2, 1):
        v = v[:half] + v[half:2 * half]
    zsq = v
    prod = jax.lax.dot_general(
        neg2emb_ref[...], zb, (((1,), (0,)), ((), ())),
        preferred_element_type=jnp.float32)
    e2b = e2_ref[...]
    sub = jax.lax.broadcasted_iota(jnp.int32, (8, CHUNK), 0)
    idx_parts = []
    minval_parts = []
    for c0 in range(0, S_BLK, CHUNK):
        zs = zsq[:, c0:c0 + CHUNK]
        cur_min = (zs + e2b[0:8]) + prod[0:8, c0:c0 + CHUNK]
        cur_row = jnp.zeros((8, CHUNK), jnp.int32)
        for r in range(1, NUM_CODES // 8):
            dr = (zs + e2b[8 * r:8 * (r + 1)]) + prod[8 * r:8 * (r + 1),
                                                      c0:c0 + CHUNK]
            lt = dr < cur_min
            cur_min = jnp.where(lt, dr, cur_min)
            cur_row = jnp.where(lt, r, cur_row)
        idx8 = cur_row * 8 + sub
        mv = jnp.min(cur_min, axis=0)
        idx_parts.append(jnp.min(
            jnp.where(cur_min == mv[None, :], idx8, NUM_CODES), axis=0))
        minval_parts.append(mv)
    idx = jnp.concatenate(idx_parts)
    minval = jnp.concatenate(minval_parts)
    idx_ref[0, 0, :] = idx

    rows = jax.lax.broadcasted_iota(jnp.int32, (NUM_CODES, S_BLK), 0)
    onehot = (rows == idx[None, :]).astype(jnp.float32)
    q_ref[0] = jax.lax.dot_general(
        emb_ref[...], onehot, (((0,), (0,)), ((), ())),
        preferred_element_type=jnp.float32)

    part = jnp.sum(minval).reshape(1, 1)
    first = (pl.program_id(0) == 0) & (pl.program_id(1) == 0)

    @pl.when(first)
    def _():
        loss_ref[...] = part

    @pl.when(jnp.logical_not(first))
    def _():
        loss_ref[...] += part


@functools.partial(jax.jit, static_argnames=())
def kernel(z, embeddings):
    B, C, T, H, W = z.shape
    S = T * H * W
    nblk = S // S_BLK
    z3 = z.reshape(B, C, S)
    e2 = jnp.sum(embeddings ** 2, axis=1)[:, None]
    neg2emb = -2.0 * embeddings

    q3, idx3, loss_sum = pl.pallas_call(
        _vq_body,
        grid=(B, nblk),
        in_specs=[
            pl.BlockSpec((1, C, S_BLK), lambda b, j: (b, 0, j)),
            pl.BlockSpec((NUM_CODES, DIM), lambda b, j: (0, 0)),
            pl.BlockSpec((NUM_CODES, DIM), lambda b, j: (0, 0)),
            pl.BlockSpec((NUM_CODES, 1), lambda b, j: (0, 0)),
        ],
        out_specs=[
            pl.BlockSpec((1, C, S_BLK), lambda b, j: (b, 0, j)),
            pl.BlockSpec((1, 1, S_BLK), lambda b, j: (b * nblk + j, 0, 0)),
            pl.BlockSpec((1, 1), lambda b, j: (0, 0)),
        ],
        out_shape=[
            jax.ShapeDtypeStruct((B, C, S), jnp.float32),
            jax.ShapeDtypeStruct((B * nblk, 1, S_BLK), jnp.int32),
            jax.ShapeDtypeStruct((1, 1), jnp.float32),
        ],
    )(z3, embeddings, neg2emb, e2)

    quantized_st = q3.reshape(B, C, T, H, W)
    encoding_indices = idx3.reshape(B, T, H, W)
    vq_loss = (1.0 + 0.25) * loss_sum[0, 0] / z.size
    return (quantized_st, vq_loss, encoding_indices)


# speedup vs baseline: 2.8682x; 1.0115x over previous
import functools

import jax
import jax.numpy as jnp
from jax.experimental import pallas as pl

NUM_CODES = 1024
DIM = 64
S_BLK = 4096
CHUNK = 512


def _vq_body(z_ref, emb_ref, neg2emb_ref, e2_ref,
             q_ref, idx_ref, loss_ref):
    zb = z_ref[0]
    v = zb * zb
    for half in (32, 16, 8, 4, 2, 1):
        v = v[:half] + v[half:2 * half]
    zsq = v
    prod = jax.lax.dot_general(
        neg2emb_ref[...], zb, (((1,), (0,)), ((), ())),
        preferred_element_type=jnp.float32)
    e2b = e2_ref[...]
    sub = jax.lax.broadcasted_iota(jnp.int32, (8, CHUNK), 0)
    idx_parts = []
    minval_parts = []
    for c0 in range(0, S_BLK, CHUNK):
        zs = zsq[:, c0:c0 + CHUNK]
        cur_min = (zs + e2b[0:8]) + prod[0:8, c0:c0 + CHUNK]
        cur_row = jnp.zeros((8, CHUNK), jnp.int32)
        for r in range(1, NUM_CODES // 8):
            dr = (zs + e2b[8 * r:8 * (r + 1)]) + prod[8 * r:8 * (r + 1),
                                                      c0:c0 + CHUNK]
            lt = dr < cur_min
            cur_min = jnp.where(lt, dr, cur_min)
            cur_row = jnp.where(lt, r, cur_row)
        idx8 = cur_row * 8 + sub
        mv = jnp.min(cur_min, axis=0)
        idx_parts.append(jnp.min(
            jnp.where(cur_min == mv[None, :], idx8, NUM_CODES), axis=0))
        minval_parts.append(mv)
    idx = jnp.concatenate(idx_parts)
    minval = jnp.concatenate(minval_parts)
    idx_ref[0, 0, :] = idx

    rows = jax.lax.broadcasted_iota(jnp.int32, (NUM_CODES, S_BLK), 0)
    onehot = (rows == idx[None, :]).astype(jnp.float32)
    q_ref[0] = jax.lax.dot_general(
        emb_ref[...], onehot, (((0,), (0,)), ((), ())),
        preferred_element_type=jnp.float32)

    part = jnp.sum(minval).reshape(1, 1)
    first = (pl.program_id(0) == 0) & (pl.program_id(1) == 0)

    @pl.when(first)
    def _():
        loss_ref[...] = part

    @pl.when(jnp.logical_not(first))
    def _():
        loss_ref[...] += part


@functools.partial(jax.jit, static_argnames=())
def kernel(z, embeddings):
    B, C, T, H, W = z.shape
    S = T * H * W
    nblk = S // S_BLK
    z3 = z.reshape(B, C, S)
    e2 = jnp.sum(embeddings ** 2, axis=1)[:, None]
    neg2emb = -2.0 * embeddings

    q3, idx3, loss_sum = pl.pallas_call(
        _vq_body,
        grid=(B, nblk),
        in_specs=[
            pl.BlockSpec((1, C, S_BLK), lambda b, j: (b, 0, j)),
            pl.BlockSpec((NUM_CODES, DIM), lambda b, j: (0, 0)),
            pl.BlockSpec((NUM_CODES, DIM), lambda b, j: (0, 0)),
            pl.BlockSpec((NUM_CODES, 1), lambda b, j: (0, 0)),
        ],
        out_specs=[
            pl.BlockSpec((1, C, S_BLK), lambda b, j: (b, 0, j)),
            pl.BlockSpec((1, 1, S_BLK), lambda b, j: (b * nblk + j, 0, 0)),
            pl.BlockSpec((1, 1), lambda b, j: (0, 0)),
        ],
        out_shape=[
            jax.ShapeDtypeStruct((B, C, S), jnp.float32),
            jax.ShapeDtypeStruct((B * nblk, 1, S_BLK), jnp.int32),
            jax.ShapeDtypeStruct((1, 1), jnp.float32),
        ],
    )(z3, embeddings, neg2emb, e2)

    quantized_st = q3.reshape(B, C, T, H, W)
    encoding_indices = idx3.reshape(B, T, H, W)
    vq_loss = (1.0 + 0.25) * loss_sum[0, 0] / z.size
    return (quantized_st, vq_loss, encoding_indices)


# speedup vs baseline: 2.8924x; 1.0084x over previous
import functools

import jax
import jax.numpy as jnp
from jax.experimental import pallas as pl

NUM_CODES = 1024
DIM = 64
S_BLK = 8192
CHUNK = 512


def _vq_body(z_ref, emb_ref, neg2emb_ref, e2_ref,
             q_ref, idx_ref, loss_ref):
    zb = z_ref[0]
    v = zb * zb
    for half in (32, 16, 8, 4, 2, 1):
        v = v[:half] + v[half:2 * half]
    zsq = v
    prod = jax.lax.dot_general(
        neg2emb_ref[...], zb, (((1,), (0,)), ((), ())),
        preferred_element_type=jnp.float32)
    e2b = e2_ref[...]
    sub = jax.lax.broadcasted_iota(jnp.int32, (8, CHUNK), 0)
    idx_parts = []
    minval_parts = []
    for c0 in range(0, S_BLK, CHUNK):
        zs = zsq[:, c0:c0 + CHUNK]
        cur_min = (zs + e2b[0:8]) + prod[0:8, c0:c0 + CHUNK]
        cur_row = jnp.zeros((8, CHUNK), jnp.int32)
        for r in range(1, NUM_CODES // 8):
            dr = (zs + e2b[8 * r:8 * (r + 1)]) + prod[8 * r:8 * (r + 1),
                                                      c0:c0 + CHUNK]
            lt = dr < cur_min
            cur_min = jnp.where(lt, dr, cur_min)
            cur_row = jnp.where(lt, r, cur_row)
        idx8 = cur_row * 8 + sub
        mv = jnp.min(cur_min, axis=0)
        idx_parts.append(jnp.min(
            jnp.where(cur_min == mv[None, :], idx8, NUM_CODES), axis=0))
        minval_parts.append(mv)
    idx = jnp.concatenate(idx_parts)
    minval = jnp.concatenate(minval_parts)
    idx_ref[0, 0, :] = idx

    rows = jax.lax.broadcasted_iota(jnp.int32, (NUM_CODES, S_BLK), 0)
    onehot = (rows == idx[None, :]).astype(jnp.float32)
    q_ref[0] = jax.lax.dot_general(
        emb_ref[...], onehot, (((0,), (0,)), ((), ())),
        preferred_element_type=jnp.float32)

    part = jnp.sum(minval).reshape(1, 1)
    first = (pl.program_id(0) == 0) & (pl.program_id(1) == 0)

    @pl.when(first)
    def _():
        loss_ref[...] = part

    @pl.when(jnp.logical_not(first))
    def _():
        loss_ref[...] += part


@functools.partial(jax.jit, static_argnames=())
def kernel(z, embeddings):
    B, C, T, H, W = z.shape
    S = T * H * W
    nblk = S // S_BLK
    z3 = z.reshape(B, C, S)
    e2 = jnp.sum(embeddings ** 2, axis=1)[:, None]
    neg2emb = -2.0 * embeddings

    q3, idx3, loss_sum = pl.pallas_call(
        _vq_body,
        grid=(B, nblk),
        in_specs=[
            pl.BlockSpec((1, C, S_BLK), lambda b, j: (b, 0, j)),
            pl.BlockSpec((NUM_CODES, DIM), lambda b, j: (0, 0)),
            pl.BlockSpec((NUM_CODES, DIM), lambda b, j: (0, 0)),
            pl.BlockSpec((NUM_CODES, 1), lambda b, j: (0, 0)),
        ],
        out_specs=[
            pl.BlockSpec((1, C, S_BLK), lambda b, j: (b, 0, j)),
            pl.BlockSpec((1, 1, S_BLK), lambda b, j: (b * nblk + j, 0, 0)),
            pl.BlockSpec((1, 1), lambda b, j: (0, 0)),
        ],
        out_shape=[
            jax.ShapeDtypeStruct((B, C, S), jnp.float32),
            jax.ShapeDtypeStruct((B * nblk, 1, S_BLK), jnp.int32),
            jax.ShapeDtypeStruct((1, 1), jnp.float32),
        ],
    )(z3, embeddings, neg2emb, e2)

    quantized_st = q3.reshape(B, C, T, H, W)
    encoding_indices = idx3.reshape(B, T, H, W)
    vq_loss = (1.0 + 0.25) * loss_sum[0, 0] / z.size
    return (quantized_st, vq_loss, encoding_indices)
